# baseline (speedup 1.0000x reference)
import functools

import jax
import jax.numpy as jnp
from jax import lax
from jax.experimental import pallas as pl
from jax.experimental.pallas import tpu as pltpu
from jax.experimental.pallas import tpu_sc as plsc

B = 16384
L = 20
D = 128
V = 1000
VPAD = 1024

NC = 2
NS = 16
NW = NC * NS
BPW = B // NW
CB = 4
NCHUNK = BPW // CB


GI = 80
RPG = GI // L
NG = BPW * L // GI


def _pool_body(idx_hbm, table_hbm, out_hbm, idx_v, rows0, rows1, out_v, sem0, sem1):
    wid = lax.axis_index("s") * NC + lax.axis_index("c")
    rows = (rows0, rows1)
    sems = (sem0, sem1)

    pltpu.sync_copy(idx_hbm.at[wid], idx_v)

    def fire(g, bsel):
        pltpu.async_copy(table_hbm.at[idx_v.at[g]], rows[bsel], sems[bsel])

    def drain(g, bsel):
        pltpu.make_async_copy(table_hbm.at[idx_v.at[g]], rows[bsel], sems[bsel]).wait()

    fire(0, 0)

    @pl.loop(0, NG, step=2)
    def _g(g):
        for bsel in range(2):
            cur = g + bsel

            @pl.when(cur + 1 < NG)
            def _():
                fire(cur + 1, (bsel + 1) % 2)

            drain(cur, bsel)
            for r in range(RPG):
                row = cur * RPG + r
                accs = [rows[bsel][r * L, pl.ds(j * 16, 16)] for j in range(D // 16)]
                for l in range(1, L):
                    accs = [
                        accs[j] + rows[bsel][r * L + l, pl.ds(j * 16, 16)]
                        for j in range(D // 16)
                    ]
                for j in range(D // 16):
                    out_v[row, pl.ds(j * 16, 16)] = accs[j] * (1.0 / L)

    pltpu.sync_copy(out_v, out_hbm.at[pl.ds(wid * BPW, BPW)])


@functools.cache
def _pool_sc():
    return pl.kernel(
        _pool_body,
        out_type=jax.ShapeDtypeStruct((B, D), jnp.float32),
        mesh=plsc.VectorSubcoreMesh(
            core_axis_name="c", subcore_axis_name="s", num_cores=NC, num_subcores=NS
        ),
        scratch_types=[
            pltpu.VMEM((NG, GI), jnp.int32),
            pltpu.VMEM((GI, D), jnp.float32),
            pltpu.VMEM((GI, D), jnp.float32),
            pltpu.VMEM((BPW, D), jnp.float32),
            pltpu.SemaphoreType.DMA,
            pltpu.SemaphoreType.DMA,
        ],
    )


def _argmax_body(x_ref, w_ref, b_ref, o_ref):
    x = x_ref[...]
    w = w_ref[...]
    logits = lax.dot_general(
        x, w, (((1,), (1,)), ((), ())), preferred_element_type=jnp.float32
    )
    logits = logits + b_ref[...]
    col = lax.broadcasted_iota(jnp.int32, logits.shape, 1)
    m = jnp.max(logits, axis=1, keepdims=True)
    o_ref[...] = jnp.min(jnp.where(logits == m, col, jnp.int32(2**30)), axis=1)


def _argmax_tc(xmean, w_pad, b_pad):
    BT = 1024
    return pl.pallas_call(
        _argmax_body,
        grid=(B // BT,),
        in_specs=[
            pl.BlockSpec((BT, D), lambda i: (i, 0)),
            pl.BlockSpec((VPAD, D), lambda i: (0, 0)),
            pl.BlockSpec((1, VPAD), lambda i: (0, 0)),
        ],
        out_specs=pl.BlockSpec((BT,), lambda i: (i,)),
        out_shape=jax.ShapeDtypeStruct((B,), jnp.int32),
    )(xmean, w_pad, b_pad)


@jax.jit
def kernel(input, emb_table, W, b):
    idx_flat = input.reshape(NW, NG, GI).astype(jnp.int32)
    xmean = _pool_sc()(idx_flat, emb_table)
    w_pad = jnp.zeros((VPAD, D), jnp.float32).at[:V].set(W)
    b_pad = jnp.full((1, VPAD), -1e30, jnp.float32).at[0, :V].set(b)
    return _argmax_tc(xmean, w_pad, b_pad)


# speedup vs baseline: 1.4985x; 1.4985x over previous
import functools

import jax
import jax.numpy as jnp
from jax import lax
from jax.experimental import pallas as pl
from jax.experimental.pallas import tpu as pltpu
from jax.experimental.pallas import tpu_sc as plsc

B = 16384
L = 20
D = 128
DH = D // 2
NJ = DH // 16
V = 1000
VPAD = 1024

NC = 2
NS = 16
NW = NC * NS
NRB = NW // 2
BPT = B // NRB
CR = 8
NCH = BPT // CR
CI = CR * L


def _pool_body(
    idx_hbm, table_hbm, out_hbm,
    table_v, idx0, idx1, outb0, outb1,
    sem_t, semi0, semi1, semo0, semo1,
):
    wid = lax.axis_index("s") * NC + lax.axis_index("c")
    half = wid % 2
    rb = wid // 2
    idxb = (idx0, idx1)
    semi = (semi0, semi1)
    outb = (outb0, outb1)
    semo = (semo0, semo1)

    tdma = pltpu.async_copy(table_hbm.at[half], table_v, sem_t)

    def fire_idx(ci, bsel):
        pltpu.async_copy(idx_hbm.at[rb, ci], idxb[bsel], semi[bsel])

    def drain_idx(ci, bsel):
        pltpu.make_async_copy(idx_hbm.at[rb, ci], idxb[bsel], semi[bsel]).wait()

    def out_slice(ci):
        return out_hbm.at[half, pl.ds(rb * BPT + ci * CR, CR)]

    fire_idx(0, 0)
    fire_idx(1, 1)
    tdma.wait()

    @pl.loop(0, NCH, step=2)
    def _c(ci):
        for bsel in range(2):
            cur = ci + bsel

            @pl.when(cur + 2 < NCH)
            def _():
                fire_idx(cur + 2, bsel)

            drain_idx(cur, bsel)

            @pl.when(cur >= 2)
            def _():
                pltpu.make_async_copy(outb[bsel], out_slice(cur - 2), semo[bsel]).wait()

            vecs = [idxb[bsel][pl.ds(k * 16, 16)] for k in range(CI // 16)]

            def gidx(p):
                return vecs[p // 16][p % 16]

            for r in range(CR):
                accs = [table_v[gidx(r * L), pl.ds(j * 16, 16)] for j in range(NJ)]
                for l in range(1, L):
                    il = gidx(r * L + l)
                    accs = [
                        accs[j] + table_v[il, pl.ds(j * 16, 16)]
                        for j in range(NJ)
                    ]
                for j in range(NJ):
                    outb[bsel][r, pl.ds(j * 16, 16)] = accs[j] * (1.0 / L)

            pltpu.async_copy(outb[bsel], out_slice(cur), semo[bsel])

    pltpu.make_async_copy(outb[0], out_slice(NCH - 2), semo[0]).wait()
    pltpu.make_async_copy(outb[1], out_slice(NCH - 1), semo[1]).wait()


@functools.cache
def _pool_sc():
    return pl.kernel(
        _pool_body,
        out_type=jax.ShapeDtypeStruct((2, B, DH), jnp.float32),
        mesh=plsc.VectorSubcoreMesh(
            core_axis_name="c", subcore_axis_name="s", num_cores=NC, num_subcores=NS
        ),
        scratch_types=[
            pltpu.VMEM((V, DH), jnp.float32),
            pltpu.VMEM((CI,), jnp.int32),
            pltpu.VMEM((CI,), jnp.int32),
            pltpu.VMEM((CR, DH), jnp.float32),
            pltpu.VMEM((CR, DH), jnp.float32),
            pltpu.SemaphoreType.DMA,
            pltpu.SemaphoreType.DMA,
            pltpu.SemaphoreType.DMA,
            pltpu.SemaphoreType.DMA,
            pltpu.SemaphoreType.DMA,
        ],
    )


def _argmax_body(x0_ref, x1_ref, w_ref, b_ref, o_ref):
    x = jnp.concatenate([x0_ref[...], x1_ref[...]], axis=1)
    logits = lax.dot_general(
        x, w_ref[...], (((1,), (1,)), ((), ())), preferred_element_type=jnp.float32
    )
    logits = logits + b_ref[...]
    col = lax.broadcasted_iota(jnp.int32, logits.shape, 1)
    m = jnp.max(logits, axis=1, keepdims=True)
    o_ref[...] = jnp.min(jnp.where(logits == m, col, jnp.int32(2**30)), axis=1)


def _argmax_tc(x0, x1, w_pad, b_pad):
    BT = 1024
    return pl.pallas_call(
        _argmax_body,
        grid=(B // BT,),
        in_specs=[
            pl.BlockSpec((BT, DH), lambda i: (i, 0)),
            pl.BlockSpec((BT, DH), lambda i: (i, 0)),
            pl.BlockSpec((VPAD, D), lambda i: (0, 0)),
            pl.BlockSpec((1, VPAD), lambda i: (0, 0)),
        ],
        out_specs=pl.BlockSpec((BT,), lambda i: (i,)),
        out_shape=jax.ShapeDtypeStruct((B,), jnp.int32),
    )(x0, x1, w_pad, b_pad)


@jax.jit
def kernel(input, emb_table, W, b):
    idx_flat = input.reshape(NRB, NCH, CI).astype(jnp.int32)
    table_r = emb_table.reshape(V, 2, DH).transpose(1, 0, 2)
    xhalves = _pool_sc()(idx_flat, table_r)
    w_pad = jnp.zeros((VPAD, D), jnp.float32).at[:V].set(W)
    b_pad = jnp.full((1, VPAD), -1e30, jnp.float32).at[0, :V].set(b)
    return _argmax_tc(xhalves[0], xhalves[1], w_pad, b_pad)


# speedup vs baseline: 1.5738x; 1.0503x over previous
import functools

import jax
import jax.numpy as jnp
from jax import lax
from jax.experimental import pallas as pl
from jax.experimental.pallas import tpu as pltpu
from jax.experimental.pallas import tpu_sc as plsc

B = 16384
L = 20
D = 128
DH = D // 2
NJ = DH // 16
V = 1000
VPAD = 1024

NC = 2
NS = 16
NW = NC * NS
NRB = NW // 2
CR = 8
CI = CR * L

NSPLIT = 2
BSP = B // NSPLIT


def _make_pool_body(nbatch):
    bpt = nbatch // NRB
    nch = bpt // CR

    def _pool_body(
        idx_hbm, table_hbm, out_hbm,
        table_v, idx0, idx1, outb0, outb1,
        sem_t, semi0, semi1, semo0, semo1,
    ):
        wid = lax.axis_index("s") * NC + lax.axis_index("c")
        half = wid % 2
        rb = wid // 2
        idxb = (idx0, idx1)
        semi = (semi0, semi1)
        outb = (outb0, outb1)
        semo = (semo0, semo1)

        tdma = pltpu.async_copy(table_hbm.at[half], table_v, sem_t)

        def fire_idx(ci, bsel):
            pltpu.async_copy(idx_hbm.at[rb, ci], idxb[bsel], semi[bsel])

        def drain_idx(ci, bsel):
            pltpu.make_async_copy(idx_hbm.at[rb, ci], idxb[bsel], semi[bsel]).wait()

        def out_slice(ci):
            return out_hbm.at[half, pl.ds(rb * bpt + ci * CR, CR)]

        fire_idx(0, 0)
        fire_idx(1, 1)
        tdma.wait()

        @pl.loop(0, nch, step=2)
        def _c(ci):
            for bsel in range(2):
                cur = ci + bsel

                @pl.when(cur + 2 < nch)
                def _():
                    fire_idx(cur + 2, bsel)

                drain_idx(cur, bsel)

                @pl.when(cur >= 2)
                def _():
                    pltpu.make_async_copy(
                        outb[bsel], out_slice(cur - 2), semo[bsel]
                    ).wait()

                vecs = [idxb[bsel][pl.ds(k * 16, 16)] for k in range(CI // 16)]

                def gidx(p):
                    return vecs[p // 16][p % 16]

                for r in range(CR):
                    accs = [
                        table_v[gidx(r * L), pl.ds(j * 16, 16)] for j in range(NJ)
                    ]
                    for l in range(1, L):
                        il = gidx(r * L + l)
                        accs = [
                            accs[j] + table_v[il, pl.ds(j * 16, 16)]
                            for j in range(NJ)
                        ]
                    for j in range(NJ):
                        outb[bsel][r, pl.ds(j * 16, 16)] = accs[j] * (1.0 / L)

                pltpu.async_copy(outb[bsel], out_slice(cur), semo[bsel])

        pltpu.make_async_copy(outb[0], out_slice(nch - 2), semo[0]).wait()
        pltpu.make_async_copy(outb[1], out_slice(nch - 1), semo[1]).wait()

    return _pool_body


@functools.cache
def _pool_sc(nbatch):
    return pl.kernel(
        _make_pool_body(nbatch),
        out_type=jax.ShapeDtypeStruct((2, nbatch, DH), jnp.float32),
        mesh=plsc.VectorSubcoreMesh(
            core_axis_name="c", subcore_axis_name="s", num_cores=NC, num_subcores=NS
        ),
        scratch_types=[
            pltpu.VMEM((V, DH), jnp.float32),
            pltpu.VMEM((CI,), jnp.int32),
            pltpu.VMEM((CI,), jnp.int32),
            pltpu.VMEM((CR, DH), jnp.float32),
            pltpu.VMEM((CR, DH), jnp.float32),
            pltpu.SemaphoreType.DMA,
            pltpu.SemaphoreType.DMA,
            pltpu.SemaphoreType.DMA,
            pltpu.SemaphoreType.DMA,
            pltpu.SemaphoreType.DMA,
        ],
    )


def _argmax_body(x0_ref, x1_ref, w_ref, b_ref, o_ref):
    x = jnp.concatenate([x0_ref[...], x1_ref[...]], axis=1)
    logits = lax.dot_general(
        x, w_ref[...], (((1,), (1,)), ((), ())), preferred_element_type=jnp.float32
    )
    logits = logits + b_ref[...]
    col = lax.broadcasted_iota(jnp.int32, logits.shape, 1)
    m = jnp.max(logits, axis=1, keepdims=True)
    o_ref[...] = jnp.min(jnp.where(logits == m, col, jnp.int32(2**30)), axis=1)


def _argmax_tc(x0, x1, w_pad, b_pad):
    BT = 1024
    nb = x0.shape[0]
    return pl.pallas_call(
        _argmax_body,
        grid=(nb // BT,),
        in_specs=[
            pl.BlockSpec((BT, DH), lambda i: (i, 0)),
            pl.BlockSpec((BT, DH), lambda i: (i, 0)),
            pl.BlockSpec((VPAD, D), lambda i: (0, 0)),
            pl.BlockSpec((1, VPAD), lambda i: (0, 0)),
        ],
        out_specs=pl.BlockSpec((BT,), lambda i: (i,)),
        out_shape=jax.ShapeDtypeStruct((nb,), jnp.int32),
    )(x0, x1, w_pad, b_pad)


@jax.jit
def kernel(input, emb_table, W, b):
    idx = input.astype(jnp.int32).reshape(NSPLIT, NRB, BSP // NRB // CR, CI)
    table_r = emb_table.reshape(V, 2, DH).transpose(1, 0, 2)
    w_pad = jnp.zeros((VPAD, D), jnp.float32).at[:V].set(W)
    b_pad = jnp.full((1, VPAD), -1e30, jnp.float32).at[0, :V].set(b)

    outs = []
    for s in range(NSPLIT):
        xh = _pool_sc(BSP)(idx[s], table_r)
        outs.append(_argmax_tc(xh[0], xh[1], w_pad, b_pad))
    return jnp.concatenate(outs)
